# baseline (speedup 1.0000x reference)
import functools

import jax
import jax.numpy as jnp
from jax import lax
from jax.experimental import pallas as pl
from jax.experimental.pallas import tpu as pltpu
from jax.experimental.pallas import tpu_sc as plsc

N = 10000
E = 320000
D = 128
B = 256
NPAD = 10240
RPT = 320
CORE_ROWS = 5120
JUNK = CORE_ROWS
AGG_ROWS = 5248
ZR = AGG_ROWS // 16
NT = 32
SCAN = 512
NSCAN = E // SCAN
EBUF = 2688
CAP = 322560
XCH = 5
XK = 64
XPT = XCH * XK

_mesh = plsc.VectorSubcoreMesh(core_axis_name="c", subcore_axis_name="s")
_HIGH = jax.lax.Precision.HIGHEST
_DEF = jax.lax.Precision.DEFAULT
i32 = jnp.int32


@functools.partial(
    pl.kernel,
    mesh=_mesh,
    out_type=jax.ShapeDtypeStruct((NPAD, D), jnp.float32),
    compiler_params=pltpu.CompilerParams(needs_layout_passes=False),
    scratch_types=[
        pltpu.VMEM((XCH, XK), jnp.int32),
        pltpu.VMEM((XPT, D), jnp.float32),
        pltpu.SemaphoreType.DMA,
    ],
)
def _sc_embed(x_hbm, emb_hbm, out_hbm, xv, rows, sem):
    c = lax.axis_index("c")
    s = lax.axis_index("s")
    w = c * 16 + s
    pltpu.sync_copy(x_hbm.at[w], xv)
    for j in range(XCH):
        pltpu.async_copy(emb_hbm.at[xv.at[j]], rows.at[pl.ds(j * XK, XK)], sem).wait()
    pltpu.sync_copy(rows, out_hbm.at[pl.ds(pl.multiple_of(w * XPT, 8), XPT)])


@functools.partial(
    pl.kernel,
    mesh=_mesh,
    out_type=(jax.ShapeDtypeStruct((NT * CAP,), jnp.int32),
              jax.ShapeDtypeStruct((NT * CAP,), jnp.int32),
              jax.ShapeDtypeStruct((NT * 16,), jnp.int32)),
    compiler_params=pltpu.CompilerParams(needs_layout_passes=False),
    scratch_types=[
        pltpu.VMEM((SCAN,), jnp.int32),
        pltpu.VMEM((SCAN,), jnp.int32),
        pltpu.VMEM((SCAN,), jnp.int32),
        pltpu.VMEM((SCAN,), jnp.int32),
        pltpu.VMEM((EBUF,), jnp.int32),
        pltpu.VMEM((EBUF,), jnp.int32),
        pltpu.VMEM((16,), jnp.int32),
        pltpu.SemaphoreType.DMA,
        pltpu.SemaphoreType.DMA,
    ],
)
def _sc_bucket(src_hbm, dst_hbm, ls_hbm, ld_hbm, nch_hbm,
               sv0, sv1, dv0, dv1, ebs, ebd, nv, sem0, sem1):
    c = lax.axis_index("c")
    s = lax.axis_index("s")
    w = c * 16 + s
    lo = w * RPT
    hi = lo + RPT
    coff = c * CORE_ROWS
    bufs = ((sv0, dv0, sem0), (sv1, dv1, sem1))

    def start(b, par):
        svb, dvb, sem = bufs[par]
        off = pl.multiple_of(b * SCAN, 8)
        pltpu.async_copy(src_hbm.at[pl.ds(off, SCAN)], svb, sem)
        pltpu.async_copy(dst_hbm.at[pl.ds(off, SCAN)], dvb, sem)

    def wait(par):
        svb, dvb, sem = bufs[par]
        pltpu.make_async_copy(src_hbm.at[pl.ds(0, SCAN)], svb, sem).wait()
        pltpu.make_async_copy(dst_hbm.at[pl.ds(0, SCAN)], dvb, sem).wait()

    def process(par, cnt):
        svb, dvb, _ = bufs[par]
        cnt_v = jnp.broadcast_to(cnt, (16,))
        for i in range(SCAN // 16):
            d16 = dvb[pl.ds(i * 16, 16)]
            s16 = svb[pl.ds(i * 16, 16)]
            m = (d16 >= lo) & (d16 < hi)
            pos = cnt_v + lax.cumsum(m.astype(i32), axis=0) - 1
            plsc.store_scatter(ebd, [pos], d16 - coff, mask=m)
            plsc.store_scatter(ebs, [pos], s16, mask=m)
            cnt_v = cnt_v + plsc.all_reduce_population_count(m)
        return jnp.max(cnt_v)

    def flush(cnt, total):
        do = cnt >= 2048

        @pl.when(do)
        def _():
            off = pl.multiple_of(w * CAP + total, 8)
            pltpu.sync_copy(ebs.at[pl.ds(0, 2048)], ls_hbm.at[pl.ds(off, 2048)])
            pltpu.sync_copy(ebd.at[pl.ds(0, 2048)], ld_hbm.at[pl.ds(off, 2048)])
            for i in range(SCAN // 16):
                ebs[pl.ds(i * 16, 16)] = ebs[pl.ds(2048 + i * 16, 16)]
                ebd[pl.ds(i * 16, 16)] = ebd[pl.ds(2048 + i * 16, 16)]

        cnt = jnp.where(do, cnt - 2048, cnt)
        total = jnp.where(do, total + 2048, total)
        return cnt, total

    start(0, 0)
    start(1, 1)

    def body(g, carry):
        cnt, total = carry
        for par in (0, 1):
            b = g * 2 + par
            wait(par)
            cnt = process(par, cnt)

            @pl.when(b + 2 < NSCAN)
            def _():
                start(b + 2, par)

            cnt, total = flush(cnt, total)
        return cnt, total

    cnt, total = lax.fori_loop(0, NSCAN // 2, body, (jnp.int32(0), jnp.int32(0)))
    wait(0)
    cnt = process(0, cnt)
    cnt, total = flush(cnt, total)
    padd = jnp.full((16,), JUNK, i32)
    pads = jnp.zeros((16,), i32)
    lane = lax.broadcasted_iota(i32, (16,), 0)
    ones = lane >= 0
    for i in range(8):
        pos = cnt + i * 16 + lane
        plsc.store_scatter(ebd, [pos], padd, mask=ones)
        plsc.store_scatter(ebs, [pos], pads, mask=ones)
    cnt_pad = (cnt + 127) & ~jnp.int32(127)
    off = pl.multiple_of(w * CAP + total, 8)
    pltpu.sync_copy(ebs, ls_hbm.at[pl.ds(off, EBUF)])
    pltpu.sync_copy(ebd, ld_hbm.at[pl.ds(off, EBUF)])
    nv[...] = jnp.broadcast_to((total + cnt_pad) // 128, (16,))
    pltpu.sync_copy(nv, nch_hbm.at[pl.ds(pl.multiple_of(w * 16, 8), 16)])


@functools.partial(
    pl.kernel,
    mesh=_mesh,
    out_type=jax.ShapeDtypeStruct((NPAD, D), jnp.float32),
    compiler_params=pltpu.CompilerParams(needs_layout_passes=False),
    scratch_types=[
        pltpu.VMEM((2048,), jnp.int32),
        pltpu.VMEM((2048,), jnp.int32),
        pltpu.VMEM((128,), jnp.int32),
        pltpu.VMEM((128, D), jnp.float32),
        pltpu.VMEM((128, D), jnp.float32),
        pltpu.VMEM_SHARED((AGG_ROWS, D), jnp.float32),
        pltpu.VMEM((16,), jnp.int32),
        pltpu.SemaphoreType.DMA,
        pltpu.SemaphoreType.DMA,
    ],
)
def _sc_segsum(h_hbm, ls_hbm, ld_hbm, nch_hbm, z_hbm, out_hbm,
               lsb, ldb, dv, rows0, rows1, aggsh, nv, sem0, sem1):
    c = lax.axis_index("c")
    s = lax.axis_index("s")
    w = c * 16 + s
    pltpu.sync_copy(z_hbm, aggsh.at[pl.ds(s * ZR, ZR)])
    pltpu.sync_copy(nch_hbm.at[pl.ds(pl.multiple_of(w * 16, 8), 16)], nv)
    plsc.subcore_barrier()
    n = jnp.max(nv[...])
    rows = (rows0, rows1)
    sems = (sem0, sem1)

    def load_block(bb):
        off = pl.multiple_of(w * CAP + bb * 2048, 8)
        pltpu.sync_copy(ls_hbm.at[pl.ds(off, 2048)], lsb)
        pltpu.sync_copy(ld_hbm.at[pl.ds(off, 2048)], ldb)

    def start_gather(j, p):
        o = pl.multiple_of((j % 16) * 128, 8)
        pltpu.async_copy(h_hbm.at[lsb.at[pl.ds(o, 128)]], rows[p], sems[p])

    def wait_gather(p):
        pltpu.make_async_copy(h_hbm.at[lsb.at[pl.ds(0, 128)]], rows[p], sems[p]).wait()

    @pl.when(n > 0)
    def _():
        load_block(0)
        start_gather(0, 0)

    def body(g, carry):
        for p in (0, 1):
            j = g * 2 + p

            @pl.when(j < n)
            def _():
                o = pl.multiple_of((j % 16) * 128, 8)
                for i in range(8):
                    dv[pl.ds(i * 16, 16)] = ldb[pl.ds(o + i * 16, 16)]
                wait_gather(p)

                @pl.when(j + 1 < n)
                def _():
                    @pl.when((j + 1) % 16 == 0)
                    def _():
                        load_block((j + 1) // 16)

                    start_gather(j + 1, 1 - p)

                pltpu.sync_copy(rows[p], aggsh.at[dv], add=True)

        return carry

    lax.fori_loop(0, (n + 1) // 2, body, 0)
    plsc.subcore_barrier()
    pltpu.sync_copy(
        aggsh.at[pl.ds(s * RPT, RPT)],
        out_hbm.at[pl.ds(pl.multiple_of(c * CORE_ROWS + s * RPT, 8), RPT)],
    )


def _dense_body(a, h, wr, br, wt, o):
    t = lax.dot_general(a[...], wr[...], (((1,), (1,)), ((), ())),
                        precision=_DEF, preferred_element_type=jnp.float32)
    t = t + br[...]
    t = t + lax.dot_general(h[...], wt[...], (((1,), (1,)), ((), ())),
                            precision=_DEF, preferred_element_type=jnp.float32)
    o[...] = jnp.where(t > 0, t, t * 0.01)


_NBLK = 1024

_dense_call = pl.pallas_call(
    _dense_body,
    grid=(NPAD // _NBLK,),
    in_specs=[
        pl.BlockSpec((_NBLK, D), lambda i: (i, 0)),
        pl.BlockSpec((_NBLK, D), lambda i: (i, 0)),
        pl.BlockSpec((D, D), lambda i: (0, 0)),
        pl.BlockSpec((1, D), lambda i: (0, 0)),
        pl.BlockSpec((D, D), lambda i: (0, 0)),
    ],
    out_specs=pl.BlockSpec((_NBLK, D), lambda i: (i, 0)),
    out_shape=jax.ShapeDtypeStruct((NPAD, D), jnp.float32),
)


def _leaky(v):
    return jnp.where(v > 0, v, v * 0.01)


def _head_body(h3, bat, w1, b1, w2, b2, w3, b3, g1, be1, g2, be2, o):
    seg = lax.broadcasted_iota(jnp.int32, (B, 1), 0)
    sums = jnp.zeros((B, D), jnp.float32)
    counts = jnp.zeros((B, 1), jnp.float32)
    for k in range(NPAD // _NBLK):
        bc = bat[:, pl.ds(k * _NBLK, _NBLK)]
        oh = (bc == seg).astype(jnp.float32)
        sums = sums + lax.dot_general(
            oh, h3[pl.ds(k * _NBLK, _NBLK), :], (((1,), (0,)), ((), ())),
            precision=_HIGH, preferred_element_type=jnp.float32)
        counts = counts + jnp.sum(oh, axis=1, keepdims=True)
    g = sums / jnp.maximum(counts, 1.0)

    t1 = lax.dot_general(g, w1[...], (((1,), (1,)), ((), ())),
                         precision=_DEF, preferred_element_type=jnp.float32) + b1[...]
    mu = jnp.mean(t1, axis=0, keepdims=True)
    var = jnp.mean((t1 - mu) ** 2, axis=0, keepdims=True)
    t1 = _leaky((t1 - mu) / jnp.sqrt(var + 1e-5) * g1[...] + be1[...])

    t2 = lax.dot_general(t1, w2[...], (((1,), (1,)), ((), ())),
                         precision=_DEF, preferred_element_type=jnp.float32) + b2[...]
    mu2 = jnp.mean(t2, axis=0, keepdims=True)
    var2 = jnp.mean((t2 - mu2) ** 2, axis=0, keepdims=True)
    t2 = _leaky((t2 - mu2) / jnp.sqrt(var2 + 1e-5) * g2[...] + be2[...])

    o[...] = jnp.sum(t2 * w3[...], axis=1, keepdims=True) + b3[...]


_head_call = pl.pallas_call(
    _head_body,
    out_shape=jax.ShapeDtypeStruct((B, 1), jnp.float32),
)


def kernel(x, edge_index, batch, emb_table,
           Wrel1, brel1, Wroot1, Wrel2, brel2, Wroot2, Wrel3, brel3, Wroot3,
           lin1_W, lin1_b, lin2_W, lin2_b, lin3_W, lin3_b,
           bn1_g, bn1_b, bn2_g, bn2_b):
    src = edge_index[0].astype(i32)
    dst = edge_index[1].astype(i32)
    xp = jnp.concatenate([x.astype(i32), jnp.zeros((NPAD - N,), i32)]).reshape(NT, XCH, XK)
    batp = jnp.concatenate([batch.astype(i32), jnp.full((NPAD - N,), B, i32)]).reshape(1, NPAD)
    zeros = jnp.zeros((ZR, D), jnp.float32)

    ls, ld, nch = _sc_bucket(src, dst)
    h = _sc_embed(xp, emb_table)
    for Wrel, brel, Wroot in ((Wrel1, brel1, Wroot1),
                              (Wrel2, brel2, Wroot2),
                              (Wrel3, brel3, Wroot3)):
        agg = _sc_segsum(h, ls, ld, nch, zeros)
        h = _dense_call(agg, h, Wrel, brel.reshape(1, D), Wroot)

    out = _head_call(h, batp,
                     lin1_W, lin1_b.reshape(1, 1024),
                     lin2_W, lin2_b.reshape(1, D),
                     lin3_W, lin3_b.reshape(1, 1),
                     bn1_g.reshape(1, 1024), bn1_b.reshape(1, 1024),
                     bn2_g.reshape(1, D), bn2_b.reshape(1, D))
    return out[:, 0]


# speedup vs baseline: 1.0058x; 1.0058x over previous
import functools

import jax
import jax.numpy as jnp
from jax import lax
from jax.experimental import pallas as pl
from jax.experimental.pallas import tpu as pltpu
from jax.experimental.pallas import tpu_sc as plsc

N = 10000
E = 320000
D = 128
B = 256
NPAD = 10240
RPT = 320
CORE_ROWS = 5120
JUNK = CORE_ROWS
AGG_ROWS = 5248
ZR = AGG_ROWS // 16
NT = 32
SCAN = 512
NSCAN = E // SCAN
EBUF = 2688
CAP = 322560
XCH = 5
XK = 64
XPT = XCH * XK

_mesh = plsc.VectorSubcoreMesh(core_axis_name="c", subcore_axis_name="s")
_HIGH = jax.lax.Precision.HIGHEST
_DEF = jax.lax.Precision.DEFAULT
i32 = jnp.int32


@functools.partial(
    pl.kernel,
    mesh=_mesh,
    out_type=jax.ShapeDtypeStruct((NPAD, D), jnp.float32),
    compiler_params=pltpu.CompilerParams(needs_layout_passes=False),
    scratch_types=[
        pltpu.VMEM((XCH, XK), jnp.int32),
        pltpu.VMEM((XPT, D), jnp.float32),
        pltpu.SemaphoreType.DMA,
    ],
)
def _sc_embed(x_hbm, emb_hbm, out_hbm, xv, rows, sem):
    c = lax.axis_index("c")
    s = lax.axis_index("s")
    w = c * 16 + s
    pltpu.sync_copy(x_hbm.at[w], xv)
    for j in range(XCH):
        pltpu.async_copy(emb_hbm.at[xv.at[j]], rows.at[pl.ds(j * XK, XK)], sem).wait()
    pltpu.sync_copy(rows, out_hbm.at[pl.ds(pl.multiple_of(w * XPT, 8), XPT)])


@functools.partial(
    pl.kernel,
    mesh=_mesh,
    out_type=(jax.ShapeDtypeStruct((NT * CAP,), jnp.int32),
              jax.ShapeDtypeStruct((NT * CAP,), jnp.int32),
              jax.ShapeDtypeStruct((NT * 16,), jnp.int32)),
    compiler_params=pltpu.CompilerParams(needs_layout_passes=False),
    scratch_types=[
        pltpu.VMEM((SCAN,), jnp.int32),
        pltpu.VMEM((SCAN,), jnp.int32),
        pltpu.VMEM((SCAN,), jnp.int32),
        pltpu.VMEM((SCAN,), jnp.int32),
        pltpu.VMEM((EBUF,), jnp.int32),
        pltpu.VMEM((EBUF,), jnp.int32),
        pltpu.VMEM((16,), jnp.int32),
        pltpu.SemaphoreType.DMA,
        pltpu.SemaphoreType.DMA,
    ],
)
def _sc_bucket(src_hbm, dst_hbm, ls_hbm, ld_hbm, nch_hbm,
               sv0, sv1, dv0, dv1, ebs, ebd, nv, sem0, sem1):
    c = lax.axis_index("c")
    s = lax.axis_index("s")
    w = c * 16 + s
    lo = w * RPT
    hi = lo + RPT
    coff = c * CORE_ROWS
    bufs = ((sv0, dv0, sem0), (sv1, dv1, sem1))

    def start(b, par):
        svb, dvb, sem = bufs[par]
        off = pl.multiple_of(b * SCAN, 8)
        pltpu.async_copy(src_hbm.at[pl.ds(off, SCAN)], svb, sem)
        pltpu.async_copy(dst_hbm.at[pl.ds(off, SCAN)], dvb, sem)

    def wait(par):
        svb, dvb, sem = bufs[par]
        pltpu.make_async_copy(src_hbm.at[pl.ds(0, SCAN)], svb, sem).wait()
        pltpu.make_async_copy(dst_hbm.at[pl.ds(0, SCAN)], dvb, sem).wait()

    def process(par, cnt):
        svb, dvb, _ = bufs[par]
        for i in range(SCAN // 16):
            d16 = dvb[pl.ds(i * 16, 16)]
            s16 = svb[pl.ds(i * 16, 16)]
            m = (d16 >= lo) & (d16 < hi)
            pos = cnt + lax.cumsum(m.astype(i32), axis=0) - 1
            plsc.store_scatter(ebd, [pos], d16 - coff, mask=m)
            plsc.store_scatter(ebs, [pos], s16, mask=m)
            cnt = cnt + jnp.sum(m.astype(i32))
        return cnt

    def flush(cnt, total):
        do = cnt >= 2048

        @pl.when(do)
        def _():
            off = pl.multiple_of(w * CAP + total, 8)
            pltpu.sync_copy(ebs.at[pl.ds(0, 2048)], ls_hbm.at[pl.ds(off, 2048)])
            pltpu.sync_copy(ebd.at[pl.ds(0, 2048)], ld_hbm.at[pl.ds(off, 2048)])
            for i in range(SCAN // 16):
                ebs[pl.ds(i * 16, 16)] = ebs[pl.ds(2048 + i * 16, 16)]
                ebd[pl.ds(i * 16, 16)] = ebd[pl.ds(2048 + i * 16, 16)]

        cnt = jnp.where(do, cnt - 2048, cnt)
        total = jnp.where(do, total + 2048, total)
        return cnt, total

    start(0, 0)
    start(1, 1)

    def body(g, carry):
        cnt, total = carry
        for par in (0, 1):
            b = g * 2 + par
            wait(par)
            cnt = process(par, cnt)

            @pl.when(b + 2 < NSCAN)
            def _():
                start(b + 2, par)

            cnt, total = flush(cnt, total)
        return cnt, total

    cnt, total = lax.fori_loop(0, NSCAN // 2, body, (jnp.int32(0), jnp.int32(0)))
    wait(0)
    cnt = process(0, cnt)
    cnt, total = flush(cnt, total)
    padd = jnp.full((16,), JUNK, i32)
    pads = jnp.zeros((16,), i32)
    lane = lax.broadcasted_iota(i32, (16,), 0)
    ones = lane >= 0
    for i in range(8):
        pos = cnt + i * 16 + lane
        plsc.store_scatter(ebd, [pos], padd, mask=ones)
        plsc.store_scatter(ebs, [pos], pads, mask=ones)
    cnt_pad = (cnt + 127) & ~jnp.int32(127)
    off = pl.multiple_of(w * CAP + total, 8)
    pltpu.sync_copy(ebs, ls_hbm.at[pl.ds(off, EBUF)])
    pltpu.sync_copy(ebd, ld_hbm.at[pl.ds(off, EBUF)])
    nv[...] = jnp.broadcast_to((total + cnt_pad) // 128, (16,))
    pltpu.sync_copy(nv, nch_hbm.at[pl.ds(pl.multiple_of(w * 16, 8), 16)])


@functools.partial(
    pl.kernel,
    mesh=_mesh,
    out_type=jax.ShapeDtypeStruct((NPAD, D), jnp.float32),
    compiler_params=pltpu.CompilerParams(needs_layout_passes=False),
    scratch_types=[
        pltpu.VMEM((2048,), jnp.int32),
        pltpu.VMEM((2048,), jnp.int32),
        pltpu.VMEM((128,), jnp.int32),
        pltpu.VMEM((128, D), jnp.float32),
        pltpu.VMEM((128, D), jnp.float32),
        pltpu.VMEM_SHARED((AGG_ROWS, D), jnp.float32),
        pltpu.VMEM((16,), jnp.int32),
        pltpu.SemaphoreType.DMA,
        pltpu.SemaphoreType.DMA,
    ],
)
def _sc_segsum(h_hbm, ls_hbm, ld_hbm, nch_hbm, z_hbm, out_hbm,
               lsb, ldb, dv, rows0, rows1, aggsh, nv, sem0, sem1):
    c = lax.axis_index("c")
    s = lax.axis_index("s")
    w = c * 16 + s
    pltpu.sync_copy(z_hbm, aggsh.at[pl.ds(s * ZR, ZR)])
    pltpu.sync_copy(nch_hbm.at[pl.ds(pl.multiple_of(w * 16, 8), 16)], nv)
    plsc.subcore_barrier()
    n = jnp.max(nv[...])
    rows = (rows0, rows1)
    sems = (sem0, sem1)

    def load_block(bb):
        off = pl.multiple_of(w * CAP + bb * 2048, 8)
        pltpu.sync_copy(ls_hbm.at[pl.ds(off, 2048)], lsb)
        pltpu.sync_copy(ld_hbm.at[pl.ds(off, 2048)], ldb)

    def start_gather(j, p):
        o = pl.multiple_of((j % 16) * 128, 8)
        pltpu.async_copy(h_hbm.at[lsb.at[pl.ds(o, 128)]], rows[p], sems[p])

    def wait_gather(p):
        pltpu.make_async_copy(h_hbm.at[lsb.at[pl.ds(0, 128)]], rows[p], sems[p]).wait()

    @pl.when(n > 0)
    def _():
        load_block(0)
        start_gather(0, 0)

    def body(g, carry):
        for p in (0, 1):
            j = g * 2 + p

            @pl.when(j < n)
            def _():
                o = pl.multiple_of((j % 16) * 128, 8)
                for i in range(8):
                    dv[pl.ds(i * 16, 16)] = ldb[pl.ds(o + i * 16, 16)]
                wait_gather(p)

                @pl.when(j + 1 < n)
                def _():
                    @pl.when((j + 1) % 16 == 0)
                    def _():
                        load_block((j + 1) // 16)

                    start_gather(j + 1, 1 - p)

                pltpu.sync_copy(rows[p], aggsh.at[dv], add=True)

        return carry

    lax.fori_loop(0, (n + 1) // 2, body, 0)
    plsc.subcore_barrier()
    pltpu.sync_copy(
        aggsh.at[pl.ds(s * RPT, RPT)],
        out_hbm.at[pl.ds(pl.multiple_of(c * CORE_ROWS + s * RPT, 8), RPT)],
    )


def _dense_body(a, h, wr, br, wt, o):
    t = lax.dot_general(a[...], wr[...], (((1,), (1,)), ((), ())),
                        precision=_DEF, preferred_element_type=jnp.float32)
    t = t + br[...]
    t = t + lax.dot_general(h[...], wt[...], (((1,), (1,)), ((), ())),
                            precision=_DEF, preferred_element_type=jnp.float32)
    o[...] = jnp.where(t > 0, t, t * 0.01)


_NBLK = 1024

_dense_call = pl.pallas_call(
    _dense_body,
    grid=(NPAD // _NBLK,),
    in_specs=[
        pl.BlockSpec((_NBLK, D), lambda i: (i, 0)),
        pl.BlockSpec((_NBLK, D), lambda i: (i, 0)),
        pl.BlockSpec((D, D), lambda i: (0, 0)),
        pl.BlockSpec((1, D), lambda i: (0, 0)),
        pl.BlockSpec((D, D), lambda i: (0, 0)),
    ],
    out_specs=pl.BlockSpec((_NBLK, D), lambda i: (i, 0)),
    out_shape=jax.ShapeDtypeStruct((NPAD, D), jnp.float32),
)


def _leaky(v):
    return jnp.where(v > 0, v, v * 0.01)


def _head_body(h3, bat, w1, b1, w2, b2, w3, b3, g1, be1, g2, be2, o):
    seg = lax.broadcasted_iota(jnp.int32, (B, 1), 0)
    sums = jnp.zeros((B, D), jnp.float32)
    counts = jnp.zeros((B, 1), jnp.float32)
    for k in range(NPAD // _NBLK):
        bc = bat[:, pl.ds(k * _NBLK, _NBLK)]
        oh = (bc == seg).astype(jnp.float32)
        sums = sums + lax.dot_general(
            oh, h3[pl.ds(k * _NBLK, _NBLK), :], (((1,), (0,)), ((), ())),
            precision=_HIGH, preferred_element_type=jnp.float32)
        counts = counts + jnp.sum(oh, axis=1, keepdims=True)
    g = sums / jnp.maximum(counts, 1.0)

    t1 = lax.dot_general(g, w1[...], (((1,), (1,)), ((), ())),
                         precision=_DEF, preferred_element_type=jnp.float32) + b1[...]
    mu = jnp.mean(t1, axis=0, keepdims=True)
    var = jnp.mean((t1 - mu) ** 2, axis=0, keepdims=True)
    t1 = _leaky((t1 - mu) / jnp.sqrt(var + 1e-5) * g1[...] + be1[...])

    t2 = lax.dot_general(t1, w2[...], (((1,), (1,)), ((), ())),
                         precision=_DEF, preferred_element_type=jnp.float32) + b2[...]
    mu2 = jnp.mean(t2, axis=0, keepdims=True)
    var2 = jnp.mean((t2 - mu2) ** 2, axis=0, keepdims=True)
    t2 = _leaky((t2 - mu2) / jnp.sqrt(var2 + 1e-5) * g2[...] + be2[...])

    o[...] = jnp.sum(t2 * w3[...], axis=1, keepdims=True) + b3[...]


_head_call = pl.pallas_call(
    _head_body,
    out_shape=jax.ShapeDtypeStruct((B, 1), jnp.float32),
)


def kernel(x, edge_index, batch, emb_table,
           Wrel1, brel1, Wroot1, Wrel2, brel2, Wroot2, Wrel3, brel3, Wroot3,
           lin1_W, lin1_b, lin2_W, lin2_b, lin3_W, lin3_b,
           bn1_g, bn1_b, bn2_g, bn2_b):
    src = edge_index[0].astype(i32)
    dst = edge_index[1].astype(i32)
    xp = jnp.concatenate([x.astype(i32), jnp.zeros((NPAD - N,), i32)]).reshape(NT, XCH, XK)
    batp = jnp.concatenate([batch.astype(i32), jnp.full((NPAD - N,), B, i32)]).reshape(1, NPAD)
    zeros = jnp.zeros((ZR, D), jnp.float32)

    ls, ld, nch = _sc_bucket(src, dst)
    h = _sc_embed(xp, emb_table)
    for Wrel, brel, Wroot in ((Wrel1, brel1, Wroot1),
                              (Wrel2, brel2, Wroot2),
                              (Wrel3, brel3, Wroot3)):
        agg = _sc_segsum(h, ls, ld, nch, zeros)
        h = _dense_call(agg, h, Wrel, brel.reshape(1, D), Wroot)

    out = _head_call(h, batp,
                     lin1_W, lin1_b.reshape(1, 1024),
                     lin2_W, lin2_b.reshape(1, D),
                     lin3_W, lin3_b.reshape(1, 1),
                     bn1_g.reshape(1, 1024), bn1_b.reshape(1, 1024),
                     bn2_g.reshape(1, D), bn2_b.reshape(1, D))
    return out[:, 0]


# speedup vs baseline: 1.0063x; 1.0005x over previous
import functools

import jax
import jax.numpy as jnp
from jax import lax
from jax.experimental import pallas as pl
from jax.experimental.pallas import tpu as pltpu
from jax.experimental.pallas import tpu_sc as plsc

N = 10000
E = 320000
D = 128
B = 256
NPAD = 10240
RPT = 320
CORE_ROWS = 5120
JUNK = CORE_ROWS
AGG_ROWS = 5248
ZR = AGG_ROWS // 16
NT = 32
SCAN = 512
NSCAN = E // SCAN
EBUF = 2688
CAP = 322560
XCH = 5
XK = 64
XPT = XCH * XK

_mesh = plsc.VectorSubcoreMesh(core_axis_name="c", subcore_axis_name="s")
_HIGH = jax.lax.Precision.HIGHEST
_DEF = jax.lax.Precision.DEFAULT
i32 = jnp.int32


@functools.partial(
    pl.kernel,
    mesh=_mesh,
    out_type=jax.ShapeDtypeStruct((NPAD, D), jnp.float32),
    compiler_params=pltpu.CompilerParams(needs_layout_passes=False),
    scratch_types=[
        pltpu.VMEM((XCH, XK), jnp.int32),
        pltpu.VMEM((XPT, D), jnp.float32),
        pltpu.SemaphoreType.DMA,
    ],
)
def _sc_embed(x_hbm, emb_hbm, out_hbm, xv, rows, sem):
    c = lax.axis_index("c")
    s = lax.axis_index("s")
    w = c * 16 + s
    pltpu.sync_copy(x_hbm.at[w], xv)
    for j in range(XCH):
        pltpu.async_copy(emb_hbm.at[xv.at[j]], rows.at[pl.ds(j * XK, XK)], sem).wait()
    pltpu.sync_copy(rows, out_hbm.at[pl.ds(pl.multiple_of(w * XPT, 8), XPT)])


@functools.partial(
    pl.kernel,
    mesh=_mesh,
    out_type=(jax.ShapeDtypeStruct((NT * CAP,), jnp.int32),
              jax.ShapeDtypeStruct((NT * CAP,), jnp.int32),
              jax.ShapeDtypeStruct((NT * 16,), jnp.int32)),
    compiler_params=pltpu.CompilerParams(needs_layout_passes=False),
    scratch_types=[
        pltpu.VMEM((SCAN,), jnp.int32),
        pltpu.VMEM((SCAN,), jnp.int32),
        pltpu.VMEM((SCAN,), jnp.int32),
        pltpu.VMEM((SCAN,), jnp.int32),
        pltpu.VMEM((EBUF,), jnp.int32),
        pltpu.VMEM((EBUF,), jnp.int32),
        pltpu.VMEM((16,), jnp.int32),
        pltpu.SemaphoreType.DMA,
        pltpu.SemaphoreType.DMA,
    ],
)
def _sc_bucket(src_hbm, dst_hbm, ls_hbm, ld_hbm, nch_hbm,
               sv0, sv1, dv0, dv1, ebs, ebd, nv, sem0, sem1):
    c = lax.axis_index("c")
    s = lax.axis_index("s")
    w = c * 16 + s
    lo = w * RPT
    hi = lo + RPT
    coff = c * CORE_ROWS
    bufs = ((sv0, dv0, sem0), (sv1, dv1, sem1))

    def start(b, par):
        svb, dvb, sem = bufs[par]
        off = pl.multiple_of(b * SCAN, 8)
        pltpu.async_copy(src_hbm.at[pl.ds(off, SCAN)], svb, sem)
        pltpu.async_copy(dst_hbm.at[pl.ds(off, SCAN)], dvb, sem)

    def wait(par):
        svb, dvb, sem = bufs[par]
        pltpu.make_async_copy(src_hbm.at[pl.ds(0, SCAN)], svb, sem).wait()
        pltpu.make_async_copy(dst_hbm.at[pl.ds(0, SCAN)], dvb, sem).wait()

    def process(par, cnt):
        svb, dvb, _ = bufs[par]
        for i in range(SCAN // 16):
            d16 = dvb[pl.ds(i * 16, 16)]
            s16 = svb[pl.ds(i * 16, 16)]
            m = (d16 >= lo) & (d16 < hi)
            pos = cnt + lax.cumsum(m.astype(i32), axis=0) - 1
            plsc.store_scatter(ebd, [pos], d16 - coff, mask=m)
            plsc.store_scatter(ebs, [pos], s16, mask=m)
            cnt = cnt + jnp.sum(m.astype(i32))
        return cnt

    def flush(cnt, total):
        do = cnt >= 2048

        @pl.when(do)
        def _():
            off = pl.multiple_of(w * CAP + total, 8)
            pltpu.sync_copy(ebs.at[pl.ds(0, 2048)], ls_hbm.at[pl.ds(off, 2048)])
            pltpu.sync_copy(ebd.at[pl.ds(0, 2048)], ld_hbm.at[pl.ds(off, 2048)])
            for i in range(SCAN // 16):
                ebs[pl.ds(i * 16, 16)] = ebs[pl.ds(2048 + i * 16, 16)]
                ebd[pl.ds(i * 16, 16)] = ebd[pl.ds(2048 + i * 16, 16)]

        cnt = jnp.where(do, cnt - 2048, cnt)
        total = jnp.where(do, total + 2048, total)
        return cnt, total

    start(0, 0)
    start(1, 1)

    def body(g, carry):
        cnt, total = carry
        for par in (0, 1):
            b = g * 2 + par
            wait(par)
            cnt = process(par, cnt)

            @pl.when(b + 2 < NSCAN)
            def _():
                start(b + 2, par)

            cnt, total = flush(cnt, total)
        return cnt, total

    cnt, total = lax.fori_loop(0, NSCAN // 2, body, (jnp.int32(0), jnp.int32(0)))
    wait(0)
    cnt = process(0, cnt)
    cnt, total = flush(cnt, total)
    padd = jnp.full((16,), JUNK, i32)
    pads = jnp.zeros((16,), i32)
    lane = lax.broadcasted_iota(i32, (16,), 0)
    ones = lane >= 0
    for i in range(8):
        pos = cnt + i * 16 + lane
        plsc.store_scatter(ebd, [pos], padd, mask=ones)
        plsc.store_scatter(ebs, [pos], pads, mask=ones)
    cnt_pad = (cnt + 127) & ~jnp.int32(127)
    off = pl.multiple_of(w * CAP + total, 8)
    pltpu.sync_copy(ebs, ls_hbm.at[pl.ds(off, EBUF)])
    pltpu.sync_copy(ebd, ld_hbm.at[pl.ds(off, EBUF)])
    nv[...] = jnp.broadcast_to((total + cnt_pad) // 128, (16,))
    pltpu.sync_copy(nv, nch_hbm.at[pl.ds(pl.multiple_of(w * 16, 8), 16)])


@functools.partial(
    pl.kernel,
    mesh=_mesh,
    out_type=jax.ShapeDtypeStruct((NPAD, D), jnp.float32),
    compiler_params=pltpu.CompilerParams(needs_layout_passes=False),
    scratch_types=[
        pltpu.VMEM((2048,), jnp.int32),
        pltpu.VMEM((2048,), jnp.int32),
        pltpu.VMEM((128,), jnp.int32),
        pltpu.VMEM((128, D), jnp.float32),
        pltpu.VMEM((128, D), jnp.float32),
        pltpu.VMEM_SHARED((AGG_ROWS, D), jnp.float32),
        pltpu.VMEM((16,), jnp.int32),
        pltpu.SemaphoreType.DMA,
        pltpu.SemaphoreType.DMA,
    ],
)
def _sc_segsum(h_hbm, ls_hbm, ld_hbm, nch_hbm, z_hbm, out_hbm,
               lsb, ldb, dv, rows0, rows1, aggsh, nv, sem0, sem1):
    c = lax.axis_index("c")
    s = lax.axis_index("s")
    w = c * 16 + s
    pltpu.sync_copy(z_hbm, aggsh.at[pl.ds(s * ZR, ZR)])
    pltpu.sync_copy(nch_hbm.at[pl.ds(pl.multiple_of(w * 16, 8), 16)], nv)
    plsc.subcore_barrier()
    n = jnp.max(nv[...])
    rows = (rows0, rows1)
    sems = (sem0, sem1)

    def load_block(bb):
        off = pl.multiple_of(w * CAP + bb * 2048, 8)
        pltpu.sync_copy(ls_hbm.at[pl.ds(off, 2048)], lsb)
        pltpu.sync_copy(ld_hbm.at[pl.ds(off, 2048)], ldb)

    def start_gather(j, p):
        o = pl.multiple_of((j % 16) * 128, 8)
        pltpu.async_copy(h_hbm.at[lsb.at[pl.ds(o, 128)]], rows[p], sems[p])

    def wait_gather(p):
        pltpu.make_async_copy(h_hbm.at[lsb.at[pl.ds(0, 128)]], rows[p], sems[p]).wait()

    @pl.when(n > 0)
    def _():
        load_block(0)
        start_gather(0, 0)

    def body(g, carry):
        for p in (0, 1):
            j = g * 2 + p

            @pl.when(j < n)
            def _():
                o = pl.multiple_of((j % 16) * 128, 8)
                for i in range(8):
                    dv[pl.ds(i * 16, 16)] = ldb[pl.ds(o + i * 16, 16)]
                wait_gather(p)

                @pl.when(j + 1 < n)
                def _():
                    @pl.when((j + 1) % 16 == 0)
                    def _():
                        load_block((j + 1) // 16)

                    start_gather(j + 1, 1 - p)

                pltpu.sync_copy(rows[p], aggsh.at[dv], add=True)

        return carry

    lax.fori_loop(0, (n + 1) // 2, body, 0)
    plsc.subcore_barrier()
    pltpu.sync_copy(
        aggsh.at[pl.ds(s * RPT, RPT)],
        out_hbm.at[pl.ds(pl.multiple_of(c * CORE_ROWS + s * RPT, 8), RPT)],
    )


def _dense_body(a, h, wr, br, wt, o):
    t = lax.dot_general(a[...], wr[...], (((1,), (1,)), ((), ())),
                        precision=_DEF, preferred_element_type=jnp.float32)
    t = t + br[...]
    t = t + lax.dot_general(h[...], wt[...], (((1,), (1,)), ((), ())),
                            precision=_DEF, preferred_element_type=jnp.float32)
    o[...] = jnp.where(t > 0, t, t * 0.01)


_NBLK = 1024

_dense_call = pl.pallas_call(
    _dense_body,
    grid=(NPAD // _NBLK,),
    in_specs=[
        pl.BlockSpec((_NBLK, D), lambda i: (i, 0)),
        pl.BlockSpec((_NBLK, D), lambda i: (i, 0)),
        pl.BlockSpec((D, D), lambda i: (0, 0)),
        pl.BlockSpec((1, D), lambda i: (0, 0)),
        pl.BlockSpec((D, D), lambda i: (0, 0)),
    ],
    out_specs=pl.BlockSpec((_NBLK, D), lambda i: (i, 0)),
    out_shape=jax.ShapeDtypeStruct((NPAD, D), jnp.float32),
)


def _leaky(v):
    return jnp.where(v > 0, v, v * 0.01)


def _head_body(h3, bat, w1, b1, w2, b2, w3, b3, g1, be1, g2, be2, o):
    seg = lax.broadcasted_iota(jnp.int32, (B, 1), 0)
    sums = jnp.zeros((B, D), jnp.float32)
    counts = jnp.zeros((B, 1), jnp.float32)
    for k in range(NPAD // _NBLK):
        bc = bat[:, pl.ds(k * _NBLK, _NBLK)]
        oh = (bc == seg).astype(jnp.float32)
        sums = sums + lax.dot_general(
            oh, h3[pl.ds(k * _NBLK, _NBLK), :], (((1,), (0,)), ((), ())),
            precision=_HIGH, preferred_element_type=jnp.float32)
        counts = counts + jnp.sum(oh, axis=1, keepdims=True)
    g = sums / jnp.maximum(counts, 1.0)

    t1 = lax.dot_general(g, w1[...], (((1,), (1,)), ((), ())),
                         precision=_DEF, preferred_element_type=jnp.float32) + b1[...]
    mu = jnp.mean(t1, axis=0, keepdims=True)
    var = jnp.mean((t1 - mu) ** 2, axis=0, keepdims=True)
    t1 = _leaky((t1 - mu) / jnp.sqrt(var + 1e-5) * g1[...] + be1[...])

    t2 = lax.dot_general(t1, w2[...], (((1,), (1,)), ((), ())),
                         precision=_DEF, preferred_element_type=jnp.float32) + b2[...]
    mu2 = jnp.mean(t2, axis=0, keepdims=True)
    var2 = jnp.mean((t2 - mu2) ** 2, axis=0, keepdims=True)
    t2 = _leaky((t2 - mu2) / jnp.sqrt(var2 + 1e-5) * g2[...] + be2[...])

    t2b = t2.astype(jnp.bfloat16).astype(jnp.float32)
    w3b = w3[...].astype(jnp.bfloat16).astype(jnp.float32)
    o[...] = jnp.sum(t2b * w3b, axis=1, keepdims=True) + b3[...]


_head_call = pl.pallas_call(
    _head_body,
    out_shape=jax.ShapeDtypeStruct((B, 1), jnp.float32),
)


def kernel(x, edge_index, batch, emb_table,
           Wrel1, brel1, Wroot1, Wrel2, brel2, Wroot2, Wrel3, brel3, Wroot3,
           lin1_W, lin1_b, lin2_W, lin2_b, lin3_W, lin3_b,
           bn1_g, bn1_b, bn2_g, bn2_b):
    src = edge_index[0].astype(i32)
    dst = edge_index[1].astype(i32)
    xp = jnp.concatenate([x.astype(i32), jnp.zeros((NPAD - N,), i32)]).reshape(NT, XCH, XK)
    batp = jnp.concatenate([batch.astype(i32), jnp.full((NPAD - N,), B, i32)]).reshape(1, NPAD)
    zeros = jnp.zeros((ZR, D), jnp.float32)

    ls, ld, nch = _sc_bucket(src, dst)
    h = _sc_embed(xp, emb_table)
    for Wrel, brel, Wroot in ((Wrel1, brel1, Wroot1),
                              (Wrel2, brel2, Wroot2),
                              (Wrel3, brel3, Wroot3)):
        agg = _sc_segsum(h, ls, ld, nch, zeros)
        h = _dense_call(agg, h, Wrel, brel.reshape(1, D), Wroot)

    out = _head_call(h, batp,
                     lin1_W, lin1_b.reshape(1, 1024),
                     lin2_W, lin2_b.reshape(1, D),
                     lin3_W, lin3_b.reshape(1, 1),
                     bn1_g.reshape(1, 1024), bn1_b.reshape(1, 1024),
                     bn2_g.reshape(1, D), bn2_b.reshape(1, D))
    return out[:, 0]
